# Initial kernel scaffold; baseline (speedup 1.0000x reference)
import jax
import jax.numpy as jnp
from jax.experimental import pallas as pl


def kernel(x, edge_index, Wl, bl, Wr, Wlin, blin, bn_g, bn_b, codebooks, W_gnn, b_gnn, W_pred, b_pred):
    raise NotImplementedError("write your pallas kernel here")


# speedup vs baseline: 6.0258x; 6.0258x over previous
import functools

import jax
import jax.numpy as jnp
from jax import lax
from jax.experimental import pallas as pl
from jax.experimental.pallas import tpu as pltpu
from jax.experimental.pallas import tpu_sc as plsc

_N = 10000
_E = 320000
_H = 128
_OUT = 40
_L = 3
_RES = 3
_CODES = 16

_NC = 2
_NS = 16
_NW = _NC * _NS
_EPW = _E // _NW
_CHUNK = 125
_NCH = _EPW // _CHUNK
_STRIPE = _N // _NS

_BN = 1000
_GRID = _N // _BN

_F32 = jnp.float32
_I32 = jnp.int32


def _sc_body(want_deg, *refs):
    if want_deg:
        (h_hbm, src_hbm, dst_hbm, z128_hbm, one16_hbm, z16_hbm,
         agg_hbm, deg_hbm, src_v, dst_v, rows_v, ones_v, degrows_v,
         agg_sh, deg_sh, sem) = refs
    else:
        (h_hbm, src_hbm, dst_hbm, z128_hbm,
         agg_hbm, src_v, dst_v, rows_v, agg_sh, sem) = refs

    cid = lax.axis_index("c")
    sid = lax.axis_index("s")
    wid = sid * _NC + cid
    base = sid * _STRIPE

    pltpu.sync_copy(src_hbm.at[wid], src_v)
    pltpu.sync_copy(dst_hbm.at[wid], dst_v)

    pltpu.sync_copy(z128_hbm, rows_v)
    for k in range(_STRIPE // _CHUNK):
        pltpu.sync_copy(rows_v, agg_sh.at[pl.ds(base + k * _CHUNK, _CHUNK)])
    if want_deg:
        pltpu.sync_copy(one16_hbm, ones_v)
        pltpu.sync_copy(z16_hbm, degrows_v)
        for k in range(_STRIPE // _CHUNK):
            pltpu.sync_copy(degrows_v, deg_sh.at[pl.ds(base + k * _CHUNK, _CHUNK)])
    plsc.subcore_barrier()

    def step(j, carry):
        pltpu.async_copy(h_hbm.at[src_v.at[j]], rows_v, sem).wait()
        pltpu.sync_copy(rows_v, agg_sh.at[dst_v.at[j]], add=True)
        if want_deg:
            pltpu.sync_copy(ones_v, deg_sh.at[dst_v.at[j]], add=True)
        return carry

    lax.fori_loop(0, _NCH, step, 0)
    plsc.subcore_barrier()

    for k in range(_STRIPE // _CHUNK):
        sl = pl.ds(base + k * _CHUNK, _CHUNK)
        pltpu.sync_copy(agg_sh.at[sl], rows_v)
        pltpu.sync_copy(rows_v, agg_hbm.at[cid].at[sl])
    if want_deg:
        for k in range(_STRIPE // _CHUNK):
            sl = pl.ds(base + k * _CHUNK, _CHUNK)
            pltpu.sync_copy(deg_sh.at[sl], degrows_v)
            pltpu.sync_copy(degrows_v, deg_hbm.at[cid].at[sl])


def _make_sc_call(want_deg):
    mesh = plsc.VectorSubcoreMesh(
        core_axis_name="c", subcore_axis_name="s",
        num_cores=_NC, num_subcores=_NS)
    if want_deg:
        out_type = (
            jax.ShapeDtypeStruct((_NC, _N, _H), _F32),
            jax.ShapeDtypeStruct((_NC, _N, 16), _F32),
        )
    else:
        out_type = jax.ShapeDtypeStruct((_NC, _N, _H), _F32)
    scratch = [
        pltpu.VMEM((_NCH, _CHUNK), _I32),
        pltpu.VMEM((_NCH, _CHUNK), _I32),
        pltpu.VMEM((_CHUNK, _H), _F32),
    ]
    if want_deg:
        scratch.append(pltpu.VMEM((_CHUNK, 16), _F32))
        scratch.append(pltpu.VMEM((_CHUNK, 16), _F32))
    scratch.append(pltpu.VMEM_SHARED((_N, _H), _F32))
    if want_deg:
        scratch.append(pltpu.VMEM_SHARED((_N, 16), _F32))
    scratch.append(pltpu.SemaphoreType.DMA)
    return pl.kernel(
        functools.partial(_sc_body, want_deg),
        out_type=out_type,
        mesh=mesh,
        scratch_types=scratch,
        compiler_params=pltpu.CompilerParams(use_tc_tiling_on_sc=False),
    )


def _mmT(a, w):
    return lax.dot_general(
        a, w, (((1,), (1,)), ((), ())),
        preferred_element_type=_F32, precision=lax.Precision.DEFAULT)


def _rownorm(v):
    ss = jnp.sum(v * v, axis=1, keepdims=True)
    return v / jnp.maximum(jnp.sqrt(ss), 1e-12)


def _tc_layer_body(final, *refs):
    if final:
        (h_ref, xl_ref, agg_ref, deg_ref, wl_ref, bl_ref, wr_ref, wlin_ref,
         blin_ref, g_ref, b_ref, cb_ref, wp_ref, bp_ref, wg_ref, bg_ref,
         h_out, xl_out, ids_out, loss_ref, pred_out, gnn_out) = refs
    else:
        (h_ref, xl_ref, agg_ref, deg_ref, wl_ref, bl_ref, wr_ref, wlin_ref,
         blin_ref, g_ref, b_ref, cb_ref,
         h_out, xl_out, ids_out, loss_ref) = refs

    h = h_ref[...]
    agg = agg_ref[0] + agg_ref[1]
    deg = deg_ref[0, :, 0:1] + deg_ref[1, :, 0:1]
    aggn = agg * (1.0 / jnp.maximum(deg, 1.0))

    out = _mmT(aggn, wl_ref[...]) + bl_ref[...] + _mmT(h, wr_ref[...])
    z = _rownorm(out) + _mmT(h, wlin_ref[...]) + blin_ref[...]
    scale = g_ref[...] * (1.0 / jnp.sqrt(jnp.float32(1.0 + 1e-5)))
    hnew = jnp.maximum(z * scale + b_ref[...], 0.0)
    h_out[...] = hnew
    xl = xl_ref[...] + hnew
    xl_out[...] = xl

    resid = hnew
    lsum = jnp.float32(0.0)
    idcols = []
    for r in range(_RES):
        cbn = _rownorm(cb_ref[r])
        rn = _rownorm(resid)
        sim = _mmT(rn, cbn)
        m = jnp.max(sim, axis=1, keepdims=True)
        io = lax.broadcasted_iota(_I32, sim.shape, 1)
        idx = jnp.min(jnp.where(sim >= m, io, _CODES), axis=1, keepdims=True)
        oh = (io == idx).astype(_F32)
        q = lax.dot_general(
            oh, cbn, (((1,), (0,)), ((), ())),
            preferred_element_type=_F32, precision=lax.Precision.HIGHEST)
        d = q - resid
        lsum = lsum + jnp.sum(d * d)
        idcols.append(idx)
        resid = resid - q
    ids_out[...] = jnp.concatenate(idcols, axis=1)

    @pl.when(pl.program_id(0) == 0)
    def _():
        loss_ref[...] = jnp.zeros((1, 1), _F32)
    loss_ref[...] += jnp.full((1, 1), lsum * jnp.float32(0.25 / (_N * _H)))

    if final:
        pred_out[...] = _mmT(xl, wp_ref[...]) + bp_ref[...]
        gnn_out[...] = _mmT(xl, wg_ref[...]) + bg_ref[...]


def _row_spec(cols):
    return pl.BlockSpec((_BN, cols), lambda i: (i, 0))


def _const_spec(shape):
    nd = len(shape)
    return pl.BlockSpec(shape, lambda i, _n=nd: (0,) * _n)


def _make_tc_layer(final):
    in_specs = [
        _row_spec(_H),
        _row_spec(_H),
        pl.BlockSpec((_NC, _BN, _H), lambda i: (0, i, 0)),
        pl.BlockSpec((_NC, _BN, 16), lambda i: (0, i, 0)),
        _const_spec((_H, _H)),
        _const_spec((1, _H)),
        _const_spec((_H, _H)),
        _const_spec((_H, _H)),
        _const_spec((1, _H)),
        _const_spec((1, _H)),
        _const_spec((1, _H)),
        _const_spec((_RES, _CODES, _H)),
    ]
    out_shape = [
        jax.ShapeDtypeStruct((_N, _H), _F32),
        jax.ShapeDtypeStruct((_N, _H), _F32),
        jax.ShapeDtypeStruct((_N, _RES), _I32),
        jax.ShapeDtypeStruct((1, 1), _F32),
    ]
    out_specs = [
        _row_spec(_H),
        _row_spec(_H),
        _row_spec(_RES),
        pl.BlockSpec((1, 1), lambda i: (0, 0)),
    ]
    if final:
        in_specs += [
            _const_spec((_OUT, _H)),
            _const_spec((1, _OUT)),
            _const_spec((16, _H)),
            _const_spec((1, 16)),
        ]
        out_shape += [
            jax.ShapeDtypeStruct((_N, _OUT), _F32),
            jax.ShapeDtypeStruct((_N, 16), _F32),
        ]
        out_specs += [
            _row_spec(_OUT),
            _row_spec(16),
        ]
    return pl.pallas_call(
        functools.partial(_tc_layer_body, final),
        grid=(_GRID,),
        in_specs=in_specs,
        out_specs=out_specs,
        out_shape=out_shape,
    )


def kernel(x, edge_index, Wl, bl, Wr, Wlin, blin, bn_g, bn_b, codebooks,
           W_gnn, b_gnn, W_pred, b_pred):
    src = edge_index[0].reshape(_NW, _NCH, _CHUNK)
    dst = edge_index[1].reshape(_NW, _NCH, _CHUNK)
    z128 = jnp.zeros((_CHUNK, _H), _F32)
    one16 = jnp.ones((_CHUNK, 16), _F32)
    z16 = jnp.zeros((_CHUNK, 16), _F32)

    sc_first = _make_sc_call(True)
    sc_rest = _make_sc_call(False)
    tc_mid = _make_tc_layer(False)
    tc_last = _make_tc_layer(True)

    wg_pad = jnp.zeros((16, _H), _F32).at[: _L * _RES].set(W_gnn)
    bg_pad = jnp.zeros((1, 16), _F32).at[0, : _L * _RES].set(b_gnn)

    h = x
    xl = jnp.zeros((_N, _H), _F32)
    losses = []
    ids = []
    deg2 = None
    for i in range(_L):
        if i == 0:
            agg2, deg2 = sc_first(h, src, dst, z128, one16, z16)
        else:
            agg2 = sc_rest(h, src, dst, z128)
        args = (h, xl, agg2, deg2, Wl[i], bl[i].reshape(1, _H), Wr[i],
                Wlin[i], blin[i].reshape(1, _H), bn_g[i].reshape(1, _H),
                bn_b[i].reshape(1, _H), codebooks[i])
        if i < _L - 1:
            h, xl, ids_i, loss_i = tc_mid(*args)
        else:
            h, xl, ids_i, loss_i, pred, gnn_pad = tc_last(
                *args, W_pred, b_pred.reshape(1, _OUT), wg_pad, bg_pad)
        losses.append(loss_i)
        ids.append(ids_i)

    total_commit = (losses[0] + losses[1] + losses[2])[0, 0]
    id_cat = jnp.concatenate(ids, axis=1)
    gnn_id = gnn_pad[:, : _L * _RES]
    return (pred, total_commit, id_cat, gnn_id)


# speedup vs baseline: 7.8782x; 1.3074x over previous
import functools

import jax
import jax.numpy as jnp
from jax import lax
from jax.experimental import pallas as pl
from jax.experimental.pallas import tpu as pltpu
from jax.experimental.pallas import tpu_sc as plsc

_N = 10000
_E = 320000
_H = 128
_OUT = 40
_L = 3
_RES = 3
_CODES = 16

_NC = 2
_NS = 16
_NW = _NC * _NS
_EPW = _E // _NW
_CHUNK = 100
_NCH = _EPW // _CHUNK
_STRIPE = _N // _NS


def _stripe_pieces():
    out, off = [], 0
    while off < _STRIPE:
        size = min(_CHUNK, _STRIPE - off)
        out.append((off, size))
        off += size
    return out

_BN = 1000
_GRID = _N // _BN

_F32 = jnp.float32
_I32 = jnp.int32


def _sc_body(h_hbm, src_hbm, dst_hbm, z128_hbm,
             agg_hbm, src_v, dst_v, rows_v, rows2_v, agg_sh, sem, sem2):
    cid = lax.axis_index("c")
    sid = lax.axis_index("s")
    wid = sid * _NC + cid
    base = sid * _STRIPE

    pltpu.sync_copy(src_hbm.at[wid], src_v)
    pltpu.sync_copy(dst_hbm.at[wid], dst_v)

    pltpu.sync_copy(z128_hbm, rows_v)
    for off, size in _stripe_pieces():
        pltpu.sync_copy(rows_v.at[pl.ds(0, size)],
                        agg_sh.at[pl.ds(base + off, size)])
    plsc.subcore_barrier()

    pltpu.async_copy(h_hbm.at[src_v.at[0]], rows_v, sem)

    def pair(p, carry):
        a = 2 * p
        pltpu.async_copy(h_hbm.at[src_v.at[a + 1]], rows2_v, sem2)
        pltpu.make_async_copy(h_hbm.at[src_v.at[a]], rows_v, sem).wait()
        pltpu.sync_copy(rows_v, agg_sh.at[dst_v.at[a]], add=True)
        nxt = jnp.minimum(a + 2, _NCH - 1)
        pltpu.async_copy(h_hbm.at[src_v.at[nxt]], rows_v, sem)
        pltpu.make_async_copy(h_hbm.at[src_v.at[a + 1]], rows2_v, sem2).wait()
        pltpu.sync_copy(rows2_v, agg_sh.at[dst_v.at[a + 1]], add=True)
        return carry

    lax.fori_loop(0, _NCH // 2, pair, 0)
    pltpu.make_async_copy(h_hbm.at[src_v.at[_NCH - 1]], rows_v, sem).wait()
    if _NCH % 2 == 1:
        pltpu.sync_copy(rows_v, agg_sh.at[dst_v.at[_NCH - 1]], add=True)
    plsc.subcore_barrier()

    for off, size in _stripe_pieces():
        sl = pl.ds(base + off, size)
        pltpu.sync_copy(agg_sh.at[sl], rows_v.at[pl.ds(0, size)])
        pltpu.sync_copy(rows_v.at[pl.ds(0, size)], agg_hbm.at[cid].at[sl])


def _make_sc_call():
    mesh = plsc.VectorSubcoreMesh(
        core_axis_name="c", subcore_axis_name="s",
        num_cores=_NC, num_subcores=_NS)
    scratch = [
        pltpu.VMEM((_NCH, _CHUNK), _I32),
        pltpu.VMEM((_NCH, _CHUNK), _I32),
        pltpu.VMEM((_CHUNK, _H), _F32),
        pltpu.VMEM((_CHUNK, _H), _F32),
        pltpu.VMEM_SHARED((_N, _H), _F32),
        pltpu.SemaphoreType.DMA,
        pltpu.SemaphoreType.DMA,
    ]
    return pl.kernel(
        _sc_body,
        out_type=jax.ShapeDtypeStruct((_NC, _N, _H), _F32),
        mesh=mesh,
        scratch_types=scratch,
        compiler_params=pltpu.CompilerParams(use_tc_tiling_on_sc=False),
    )


def _deg_body(dst_hbm, one16_hbm, z16_hbm, deg_hbm,
              dst_v, ones_v, degrows_v, deg_sh, sem):
    cid = lax.axis_index("c")
    sid = lax.axis_index("s")
    wid = sid * _NC + cid
    base = sid * _STRIPE

    pltpu.sync_copy(dst_hbm.at[wid], dst_v)
    pltpu.sync_copy(one16_hbm, ones_v)
    pltpu.sync_copy(z16_hbm, degrows_v)
    for off, size in _stripe_pieces():
        pltpu.sync_copy(degrows_v.at[pl.ds(0, size)],
                        deg_sh.at[pl.ds(base + off, size)])
    plsc.subcore_barrier()

    def fire(j, carry):
        pltpu.async_copy(ones_v, deg_sh.at[dst_v.at[j]], sem, add=True)
        return carry

    lax.fori_loop(0, _NCH, fire, 0)

    def drain(j, carry):
        pltpu.make_async_copy(ones_v, deg_sh.at[dst_v.at[0]], sem).wait()
        return carry

    lax.fori_loop(0, _NCH, drain, 0)
    plsc.subcore_barrier()

    for off, size in _stripe_pieces():
        sl = pl.ds(base + off, size)
        pltpu.sync_copy(deg_sh.at[sl], degrows_v.at[pl.ds(0, size)])
        pltpu.sync_copy(degrows_v.at[pl.ds(0, size)], deg_hbm.at[cid].at[sl])


def _make_deg_call():
    mesh = plsc.VectorSubcoreMesh(
        core_axis_name="c", subcore_axis_name="s",
        num_cores=_NC, num_subcores=_NS)
    scratch = [
        pltpu.VMEM((_NCH, _CHUNK), _I32),
        pltpu.VMEM((_CHUNK, 16), _F32),
        pltpu.VMEM((_CHUNK, 16), _F32),
        pltpu.VMEM_SHARED((_N, 16), _F32),
        pltpu.SemaphoreType.DMA,
    ]
    return pl.kernel(
        _deg_body,
        out_type=jax.ShapeDtypeStruct((_NC, _N, 16), _F32),
        mesh=mesh,
        scratch_types=scratch,
        compiler_params=pltpu.CompilerParams(use_tc_tiling_on_sc=False),
    )


def _mmT(a, w):
    return lax.dot_general(
        a, w, (((1,), (1,)), ((), ())),
        preferred_element_type=_F32, precision=lax.Precision.DEFAULT)


def _rownorm(v):
    ss = jnp.sum(v * v, axis=1, keepdims=True)
    return v / jnp.maximum(jnp.sqrt(ss), 1e-12)


def _tc_layer_body(final, *refs):
    if final:
        (h_ref, xl_ref, agg_ref, deg_ref, wl_ref, bl_ref, wr_ref, wlin_ref,
         blin_ref, g_ref, b_ref, cb_ref, wp_ref, bp_ref, wg_ref, bg_ref,
         h_out, xl_out, ids_out, loss_ref, pred_out, gnn_out) = refs
    else:
        (h_ref, xl_ref, agg_ref, deg_ref, wl_ref, bl_ref, wr_ref, wlin_ref,
         blin_ref, g_ref, b_ref, cb_ref,
         h_out, xl_out, ids_out, loss_ref) = refs

    h = h_ref[...]
    agg = agg_ref[0] + agg_ref[1]
    deg = deg_ref[0, :, 0:1] + deg_ref[1, :, 0:1]
    aggn = agg * (1.0 / jnp.maximum(deg, 1.0))

    out = _mmT(aggn, wl_ref[...]) + bl_ref[...] + _mmT(h, wr_ref[...])
    z = _rownorm(out) + _mmT(h, wlin_ref[...]) + blin_ref[...]
    scale = g_ref[...] * (1.0 / jnp.sqrt(jnp.float32(1.0 + 1e-5)))
    hnew = jnp.maximum(z * scale + b_ref[...], 0.0)
    h_out[...] = hnew
    xl = xl_ref[...] + hnew
    xl_out[...] = xl

    resid = hnew
    lsum = jnp.float32(0.0)
    idcols = []
    for r in range(_RES):
        cbn = _rownorm(cb_ref[r])
        rn = _rownorm(resid)
        sim = _mmT(rn, cbn)
        m = jnp.max(sim, axis=1, keepdims=True)
        io = lax.broadcasted_iota(_I32, sim.shape, 1)
        idx = jnp.min(jnp.where(sim >= m, io, _CODES), axis=1, keepdims=True)
        oh = (io == idx).astype(_F32)
        q = lax.dot_general(
            oh, cbn, (((1,), (0,)), ((), ())),
            preferred_element_type=_F32, precision=lax.Precision.HIGHEST)
        d = q - resid
        lsum = lsum + jnp.sum(d * d)
        idcols.append(idx)
        resid = resid - q
    ids_out[...] = jnp.concatenate(idcols, axis=1)

    @pl.when(pl.program_id(0) == 0)
    def _():
        loss_ref[...] = jnp.zeros((1, 1), _F32)
    loss_ref[...] += jnp.full((1, 1), lsum * jnp.float32(0.25 / (_N * _H)))

    if final:
        pred_out[...] = _mmT(xl, wp_ref[...]) + bp_ref[...]
        gnn_out[...] = _mmT(xl, wg_ref[...]) + bg_ref[...]


def _row_spec(cols):
    return pl.BlockSpec((_BN, cols), lambda i: (i, 0))


def _const_spec(shape):
    nd = len(shape)
    return pl.BlockSpec(shape, lambda i, _n=nd: (0,) * _n)


def _make_tc_layer(final):
    in_specs = [
        _row_spec(_H),
        _row_spec(_H),
        pl.BlockSpec((_NC, _BN, _H), lambda i: (0, i, 0)),
        pl.BlockSpec((_NC, _BN, 16), lambda i: (0, i, 0)),
        _const_spec((_H, _H)),
        _const_spec((1, _H)),
        _const_spec((_H, _H)),
        _const_spec((_H, _H)),
        _const_spec((1, _H)),
        _const_spec((1, _H)),
        _const_spec((1, _H)),
        _const_spec((_RES, _CODES, _H)),
    ]
    out_shape = [
        jax.ShapeDtypeStruct((_N, _H), _F32),
        jax.ShapeDtypeStruct((_N, _H), _F32),
        jax.ShapeDtypeStruct((_N, _RES), _I32),
        jax.ShapeDtypeStruct((1, 1), _F32),
    ]
    out_specs = [
        _row_spec(_H),
        _row_spec(_H),
        _row_spec(_RES),
        pl.BlockSpec((1, 1), lambda i: (0, 0)),
    ]
    if final:
        in_specs += [
            _const_spec((_OUT, _H)),
            _const_spec((1, _OUT)),
            _const_spec((16, _H)),
            _const_spec((1, 16)),
        ]
        out_shape += [
            jax.ShapeDtypeStruct((_N, _OUT), _F32),
            jax.ShapeDtypeStruct((_N, 16), _F32),
        ]
        out_specs += [
            _row_spec(_OUT),
            _row_spec(16),
        ]
    return pl.pallas_call(
        functools.partial(_tc_layer_body, final),
        grid=(_GRID,),
        in_specs=in_specs,
        out_specs=out_specs,
        out_shape=out_shape,
    )


def kernel(x, edge_index, Wl, bl, Wr, Wlin, blin, bn_g, bn_b, codebooks,
           W_gnn, b_gnn, W_pred, b_pred):
    src = edge_index[0].reshape(_NW, _NCH, _CHUNK)
    dst = edge_index[1].reshape(_NW, _NCH, _CHUNK)
    z128 = jnp.zeros((_CHUNK, _H), _F32)
    one16 = jnp.ones((_CHUNK, 16), _F32)
    z16 = jnp.zeros((_CHUNK, 16), _F32)

    sc_agg = _make_sc_call()
    sc_deg = _make_deg_call()
    tc_mid = _make_tc_layer(False)
    tc_last = _make_tc_layer(True)

    wg_pad = jnp.zeros((16, _H), _F32).at[: _L * _RES].set(W_gnn)
    bg_pad = jnp.zeros((1, 16), _F32).at[0, : _L * _RES].set(b_gnn)

    h = x
    xl = jnp.zeros((_N, _H), _F32)
    losses = []
    ids = []
    deg2 = sc_deg(dst, one16, z16)
    for i in range(_L):
        agg2 = sc_agg(h, src, dst, z128)
        args = (h, xl, agg2, deg2, Wl[i], bl[i].reshape(1, _H), Wr[i],
                Wlin[i], blin[i].reshape(1, _H), bn_g[i].reshape(1, _H),
                bn_b[i].reshape(1, _H), codebooks[i])
        if i < _L - 1:
            h, xl, ids_i, loss_i = tc_mid(*args)
        else:
            h, xl, ids_i, loss_i, pred, gnn_pad = tc_last(
                *args, W_pred, b_pred.reshape(1, _OUT), wg_pad, bg_pad)
        losses.append(loss_i)
        ids.append(ids_i)

    total_commit = (losses[0] + losses[1] + losses[2])[0, 0]
    id_cat = jnp.concatenate(ids, axis=1)
    gnn_id = gnn_pad[:, : _L * _RES]
    return (pred, total_commit, id_cat, gnn_id)
